# baseline (speedup 1.0000x reference)
import functools

import jax
import jax.numpy as jnp
from jax import lax
from jax.experimental import pallas as pl
from jax.experimental.pallas import tpu as pltpu
from jax.experimental.pallas import tpu_sc as plsc

NUM_CORES = 2
NUM_SUBCORES = 16
NUM_WORKERS = NUM_CORES * NUM_SUBCORES
SUB = 4


def _full(v):
    return jnp.full((16,), v, jnp.int32)


@functools.lru_cache(maxsize=None)
def _make_gather(B, D):
    b_per_w = B // NUM_WORKERS
    n_groups = b_per_w // 16
    mesh = plsc.VectorSubcoreMesh(core_axis_name="c", subcore_axis_name="s")

    @functools.partial(
        pl.kernel,
        mesh=mesh,
        out_type=jax.ShapeDtypeStruct((D, B), jnp.float32),
        scratch_types=[
            pltpu.VMEM((b_per_w,), jnp.int32),
            pltpu.VMEM((2, SUB, D, 128), jnp.float32),
            pltpu.VMEM((D, b_per_w), jnp.float32),
            pltpu.HBM((SUB, D, 128), jnp.float32),
            pltpu.SemaphoreType.DMA,
            pltpu.SemaphoreType.DMA,
        ],
        compiler_params=pltpu.CompilerParams(
            use_tc_tiling_on_sc=True,
            needs_layout_passes=False,
            disable_bounds_checks=True,
        ),
    )
    def gather_kernel(tab_hbm, idx_hbm, out_hbm, idx_v, buf, stage, dummy, s0, s1):
        wid = lax.axis_index("s") * NUM_CORES + lax.axis_index("c")
        base = wid * b_per_w
        pltpu.sync_copy(idx_hbm.at[wid], idx_v)
        sems = (s0, s1)
        lane = lax.iota(jnp.int32, 16)

        def issue(ti_vec, sc, b):
            for l in range(SUB):
                start = pl.multiple_of(ti_vec[sc * SUB + l] * 128, 128)
                pltpu.async_copy(
                    tab_hbm.at[:, pl.ds(start, 128)],
                    buf.at[b, l],
                    sems[b],
                )

        def drain_extract(li_vec, g, sc, b, mask):
            pltpu.make_async_copy(dummy, buf.at[b], sems[b]).wait()
            for l in range(SUB):
                li = li_vec[sc * SUB + l]
                p = g * 16 + sc * SUB + l
                for cg in range(D // 16):
                    cvec = lane + cg * 16
                    val = plsc.load_gather(
                        buf, [_full(b), _full(l), cvec, _full(li)]
                    )
                    plsc.store_scatter(stage, [cvec, _full(p)], val, mask=mask)

        def group_body(g, prev_vec):
            vec = idx_v[pl.ds(g * 16, 16)]
            ti_vec = lax.shift_right_logical(vec, 7)
            li_vec = jnp.bitwise_and(vec, 127)
            prev_li = jnp.bitwise_and(prev_vec, 127)
            live = jnp.full((16,), True, jnp.bool_)
            first = g == 0
            issue(ti_vec, 0, 0)
            drain_extract(prev_li, g - 1, 3, 1, jnp.full((16,), g > 0, jnp.bool_))
            issue(ti_vec, 1, 1)
            drain_extract(li_vec, g, 0, 0, live)
            issue(ti_vec, 2, 0)
            drain_extract(li_vec, g, 1, 1, live)
            issue(ti_vec, 3, 1)
            drain_extract(li_vec, g, 2, 0, live)
            return vec

        zeros = jnp.zeros((16,), jnp.int32)
        issue(zeros, 0, 1)
        last_vec = lax.fori_loop(0, n_groups, group_body, zeros)
        drain_extract(
            jnp.bitwise_and(last_vec, 127),
            n_groups - 1,
            3,
            1,
            jnp.full((16,), True, jnp.bool_),
        )
        pltpu.sync_copy(stage, out_hbm.at[:, pl.ds(base, b_per_w)])

    return gather_kernel


def kernel(x, embed_weight):
    (B,) = x.shape
    V, D = embed_weight.shape
    tab_t = embed_weight.T
    idx = x.astype(jnp.int32).reshape(NUM_WORKERS, B // NUM_WORKERS)
    out_t = _make_gather(B, D)(tab_t, idx)
    return out_t.T[None]
